# Initial kernel scaffold; baseline (speedup 1.0000x reference)
import jax
import jax.numpy as jnp
from jax.experimental import pallas as pl


def kernel(pred_positions, target_positions):
    raise NotImplementedError("write your pallas kernel here")


# speedup vs baseline: 1.3835x; 1.3835x over previous
import jax
import jax.numpy as jnp
from jax.experimental import pallas as pl
from jax.experimental.pallas import tpu as pltpu

N = 8192
M = 8192
BJ = 1024


def _chamfer_kernel(x_ref, yt_ref, out_ref, rowmin_ref, colacc_ref):
    j = pl.program_id(0)
    nj = pl.num_programs(0)

    @pl.when(j == 0)
    def _init():
        rowmin_ref[...] = jnp.full_like(rowmin_ref, jnp.inf)
        colacc_ref[0, 0] = 0.0

    x = x_ref[...]
    yt = yt_ref[...]
    xn = jnp.sum(x * x, axis=1, keepdims=True)
    yn = jnp.sum(yt * yt, axis=0, keepdims=True)
    d = xn + yn - 2.0 * jnp.dot(x, yt)

    rowmin_ref[...] = jnp.minimum(rowmin_ref[...], jnp.min(d, axis=1, keepdims=True))
    colacc_ref[0, 0] += jnp.sum(jnp.min(d, axis=0))

    @pl.when(j == nj - 1)
    def _finish():
        mean_row = jnp.sum(rowmin_ref[...]) / N
        mean_col = colacc_ref[0, 0] / M
        out_ref[...] = jnp.full((1, 1), (mean_row + mean_col) * 0.5, jnp.float32)


def kernel(pred_positions, target_positions):
    yt = target_positions.T
    out = pl.pallas_call(
        _chamfer_kernel,
        grid=(M // BJ,),
        in_specs=[
            pl.BlockSpec((N, 3), lambda j: (0, 0)),
            pl.BlockSpec((3, BJ), lambda j: (0, j)),
        ],
        out_specs=pl.BlockSpec((1, 1), lambda j: (0, 0)),
        out_shape=jax.ShapeDtypeStruct((1, 1), jnp.float32),
        scratch_shapes=[
            pltpu.VMEM((N, 1), jnp.float32),
            pltpu.SMEM((1, 1), jnp.float32),
        ],
    )(pred_positions, yt)
    return out[0, 0]


# speedup vs baseline: 1.5743x; 1.1379x over previous
import jax
import jax.numpy as jnp
from jax.experimental import pallas as pl
from jax.experimental.pallas import tpu as pltpu

N = 8192
M = 8192
BJ = 1024


def _chamfer_kernel(x_ref, yt_ref, out_ref, xm_ref, xn_ref, rowmin_ref,
                    colacc_ref):
    j = pl.program_id(0)
    nj = pl.num_programs(0)

    @pl.when(j == 0)
    def _init():
        x = x_ref[...]
        xm_ref[...] = x * -2.0
        xn_ref[...] = jnp.sum(x * x, axis=1, keepdims=True)
        rowmin_ref[...] = jnp.full_like(rowmin_ref, jnp.inf)
        colacc_ref[0, 0] = 0.0

    yt = yt_ref[...]
    yn = jnp.sum(yt * yt, axis=0, keepdims=True)
    d = (xn_ref[...] + yn) + jnp.dot(xm_ref[...], yt)

    rowmin_ref[...] = jnp.minimum(rowmin_ref[...], jnp.min(d, axis=1, keepdims=True))
    colacc_ref[0, 0] += jnp.sum(jnp.min(d, axis=0))

    @pl.when(j == nj - 1)
    def _finish():
        mean_row = jnp.sum(rowmin_ref[...]) / N
        mean_col = colacc_ref[0, 0] / M
        out_ref[...] = jnp.full((1, 1), (mean_row + mean_col) * 0.5, jnp.float32)


def kernel(pred_positions, target_positions):
    yt = target_positions.T
    out = pl.pallas_call(
        _chamfer_kernel,
        grid=(M // BJ,),
        in_specs=[
            pl.BlockSpec((N, 3), lambda j: (0, 0)),
            pl.BlockSpec((3, BJ), lambda j: (0, j)),
        ],
        out_specs=pl.BlockSpec((1, 1), lambda j: (0, 0)),
        out_shape=jax.ShapeDtypeStruct((1, 1), jnp.float32),
        scratch_shapes=[
            pltpu.VMEM((N, 3), jnp.float32),
            pltpu.VMEM((N, 1), jnp.float32),
            pltpu.VMEM((N, 1), jnp.float32),
            pltpu.SMEM((1, 1), jnp.float32),
        ],
    )(pred_positions, yt)
    return out[0, 0]
